# Initial kernel scaffold; baseline (speedup 1.0000x reference)
import jax
import jax.numpy as jnp
from jax.experimental import pallas as pl


def kernel(x_vals, x_cons, edge_attr_v2c, edge_attr_c2v, edge_index_v2c, edge_index_c2v, batch_vals, batch_cons, W_msg_v2c, W_root_v2c, b_v2c, W_msg_c2v, W_root_c2v, b_c2v):
    raise NotImplementedError("write your pallas kernel here")


# speedup vs baseline: 2.4684x; 2.4684x over previous
import functools

import jax
import jax.numpy as jnp
from jax import lax
from jax.experimental import pallas as pl
from jax.experimental.pallas import tpu as pltpu
from jax.experimental.pallas import tpu_sc as plsc

N = 10000
E_TOT = 320000
D = 128
NC, NS = 2, 16
NW = NC * NS
EDGES_PER_W = E_TOT // NW
K = 80
CHUNKS = EDGES_PER_W // K
ACC_ROWS = 10240
ROWS_PER_S = ACC_ROWS // NS


def _sc_edge_body(p_hbm, q_hbm, src_hbm, dst_hbm, part_out, cnt_out,
                  src_v, dst_v, q_v, p_v, ones_v, zrow_v, zcnt_v,
                  acc_sh, cnt_sh, sem):
    c = lax.axis_index("c")
    s = lax.axis_index("s")
    zero16 = jnp.zeros((16,), jnp.float32)
    one16 = jnp.ones((16,), jnp.float32)

    def _ones_fill(i, carry):
        ones_v[pl.ds(i * 16, 16)] = one16
        return carry
    lax.fori_loop(0, K // 16, _ones_fill, 0)

    def _zrow_fill(i, carry):
        for j in range(D // 16):
            zrow_v[i, pl.ds(j * 16, 16)] = zero16
        return carry
    lax.fori_loop(0, 128, _zrow_fill, 0)

    def _zcnt_fill(i, carry):
        zcnt_v[pl.ds(i * 16, 16)] = zero16
        return carry
    lax.fori_loop(0, ROWS_PER_S // 16, _zcnt_fill, 0)

    for i in range(ROWS_PER_S // 128):
        pltpu.sync_copy(zrow_v, acc_sh.at[pl.ds(s * ROWS_PER_S + i * 128, 128)])
    pltpu.sync_copy(zcnt_v, cnt_sh.at[pl.ds(s * ROWS_PER_S, ROWS_PER_S)])
    plsc.subcore_barrier()

    base0 = (c * NS + s) * EDGES_PER_W

    def _chunk(k, carry):
        base = base0 + k * K
        pltpu.sync_copy(src_hbm.at[pl.ds(base, K)], src_v)
        pltpu.sync_copy(q_hbm.at[pl.ds(base, K)], q_v)
        pltpu.async_copy(p_hbm.at[src_v], p_v, sem).wait()

        def _edge(e, cin):
            for j in range(D // 16):
                sl = pl.ds(j * 16, 16)
                q_v[e, sl] = jnp.maximum(q_v[e, sl] + p_v[e, sl], zero16)
            return cin
        lax.fori_loop(0, K, _edge, 0)

        pltpu.sync_copy(dst_hbm.at[pl.ds(base, K)], dst_v)
        pltpu.sync_copy(q_v, acc_sh.at[dst_v], add=True)
        pltpu.sync_copy(ones_v, cnt_sh.at[dst_v], add=True)
        return carry
    lax.fori_loop(0, CHUNKS, _chunk, 0)
    plsc.subcore_barrier()

    pltpu.sync_copy(acc_sh.at[pl.ds(s * ROWS_PER_S, ROWS_PER_S)],
                    part_out.at[c, pl.ds(s * ROWS_PER_S, ROWS_PER_S)])
    pltpu.sync_copy(cnt_sh.at[pl.ds(s * ROWS_PER_S, ROWS_PER_S)],
                    cnt_out.at[c, pl.ds(s * ROWS_PER_S, ROWS_PER_S)])


_sc_edge_pass = functools.partial(
    pl.kernel,
    mesh=plsc.VectorSubcoreMesh(core_axis_name="c", subcore_axis_name="s"),
    out_type=[jax.ShapeDtypeStruct((NC, ACC_ROWS, D), jnp.float32),
              jax.ShapeDtypeStruct((NC, ACC_ROWS), jnp.float32)],
    scratch_types=[
        pltpu.VMEM((K,), jnp.int32),
        pltpu.VMEM((K,), jnp.int32),
        pltpu.VMEM((K, D), jnp.float32),
        pltpu.VMEM((K, D), jnp.float32),
        pltpu.VMEM((K,), jnp.float32),
        pltpu.VMEM((128, D), jnp.float32),
        pltpu.VMEM((ROWS_PER_S,), jnp.float32),
        pltpu.VMEM_SHARED((ACC_ROWS, D), jnp.float32),
        pltpu.VMEM_SHARED((ACC_ROWS,), jnp.float32),
        pltpu.SemaphoreType.DMA,
    ],
)(_sc_edge_body)


def _mm_block(x_ref, w_ref, b_ref, o_ref):
    o_ref[...] = jnp.dot(x_ref[...], w_ref[...],
                         preferred_element_type=jnp.float32) + b_ref[...]


def _matmul(x, w, b, bm):
    m, kdim = x.shape
    n = w.shape[1]
    return pl.pallas_call(
        _mm_block,
        grid=(m // bm,),
        in_specs=[pl.BlockSpec((bm, kdim), lambda i: (i, 0)),
                  pl.BlockSpec((kdim, n), lambda i: (0, 0)),
                  pl.BlockSpec((1, n), lambda i: (0, 0))],
        out_specs=pl.BlockSpec((bm, n), lambda i: (i, 0)),
        out_shape=jax.ShapeDtypeStruct((m, n), jnp.float32),
    )(x, w, b.reshape(1, n))


def _combine_block(r_ref, p0_ref, p1_ref, c0_ref, c1_ref, o_ref):
    cnt = jnp.maximum(c0_ref[...] + c1_ref[...], 1.0)
    agg = (p0_ref[...] + p1_ref[...]) / cnt
    o_ref[...] = jnp.maximum(r_ref[...] + agg, 0.0)


def _combine(r, parts, cnts, bm=1000):
    c0 = cnts[0].reshape(ACC_ROWS, 1)
    c1 = cnts[1].reshape(ACC_ROWS, 1)
    return pl.pallas_call(
        _combine_block,
        grid=(N // bm,),
        in_specs=[pl.BlockSpec((bm, D), lambda i: (i, 0)),
                  pl.BlockSpec((bm, D), lambda i: (i, 0)),
                  pl.BlockSpec((bm, D), lambda i: (i, 0)),
                  pl.BlockSpec((bm, 1), lambda i: (i, 0)),
                  pl.BlockSpec((bm, 1), lambda i: (i, 0))],
        out_specs=pl.BlockSpec((bm, D), lambda i: (i, 0)),
        out_shape=jax.ShapeDtypeStruct((N, D), jnp.float32),
    )(r, parts[0], parts[1], c0, c1)


def kernel(x_vals, x_cons, edge_attr_v2c, edge_attr_c2v,
           edge_index_v2c, edge_index_c2v, batch_vals, batch_cons,
           W_msg_v2c, W_root_v2c, b_v2c, W_msg_c2v, W_root_c2v, b_c2v):
    del batch_vals, batch_cons
    zb = jnp.zeros((D,), jnp.float32)
    e1 = jnp.pad(edge_attr_v2c, ((0, 0), (0, 4)))
    e2 = jnp.pad(edge_attr_c2v, ((0, 0), (0, 4)))
    B1 = jnp.pad(W_msg_v2c[D:], ((0, 4), (0, 0)))
    B2 = jnp.pad(W_msg_c2v[D:], ((0, 4), (0, 0)))

    P1 = _matmul(x_vals, W_msg_v2c[:D], zb, 1000)
    Q1 = _matmul(e1, B1, zb, 2000)
    R1 = _matmul(x_cons, W_root_v2c, b_v2c, 1000)
    Q2 = _matmul(e2, B2, zb, 2000)
    R2 = _matmul(x_vals, W_root_c2v, b_c2v, 1000)

    src1 = edge_index_v2c[0].astype(jnp.int32)
    dst1 = edge_index_v2c[1].astype(jnp.int32)
    src2 = edge_index_c2v[0].astype(jnp.int32)
    dst2 = edge_index_c2v[1].astype(jnp.int32)

    part1, cnt1 = _sc_edge_pass(P1, Q1, src1, dst1)
    x_cons_new = _combine(R1, part1, cnt1)

    P2 = _matmul(x_cons_new, W_msg_c2v[:D], zb, 1000)
    part2, cnt2 = _sc_edge_pass(P2, Q2, src2, dst2)
    x_vals_new = _combine(R2, part2, cnt2)

    return (x_vals_new, x_cons_new)
